# baseline (speedup 1.0000x reference)
import functools

import jax
import jax.numpy as jnp
from jax import lax
from jax.experimental import pallas as pl
from jax.experimental.pallas import tpu as pltpu
from jax.experimental.pallas import tpu_sc as plsc

LEN_RULES = 100000
NUM_REL_RULES = 50000
BATCH = 16384
HIST = 200

NC, NS, L = 2, 16, 16
NW = NC * NS

G_PAD = 100352
S_CHUNK = G_PAD // NS
S_HALF = S_CHUNK // 2
LAST_OFF = (NS - 1) * S_CHUNK
LAST_N = LEN_RULES - LAST_OFF
ROWS_W = BATCH // NW
GROUPS = ROWS_W // L
NBUF = 3

_mesh = plsc.VectorSubcoreMesh(core_axis_name="c", subcore_axis_name="s")
_params = pltpu.CompilerParams(needs_layout_passes=False)


@functools.partial(
    pl.kernel,
    out_type=jax.ShapeDtypeStruct((BATCH,), jnp.float32),
    mesh=_mesh,
    scratch_types=[
        pltpu.VMEM((G_PAD,), jnp.float32),
        pltpu.VMEM((NBUF * L, HIST), jnp.int32),
        pltpu.VMEM((S_CHUNK,), jnp.int32),
        pltpu.VMEM((S_HALF,), jnp.float32),
        pltpu.VMEM((ROWS_W,), jnp.float32),
        pltpu.VMEM_SHARED((G_PAD,), jnp.float32),
        [pltpu.SemaphoreType.DMA] * NBUF,
    ],
    compiler_params=_params,
)
def _run(rules_hbm, g2l_hbm, table_hbm, out_hbm,
         tab_v, idx_v, g2l_v, fchunk_v, out_v, fused_sh, sems):
    s = lax.axis_index("s")
    w = s * NC + lax.axis_index("c")
    row0 = w * ROWS_W
    iota = lax.iota(jnp.int32, L)
    zeros = jnp.zeros((L,), jnp.int32)

    def blk_src(g):
        return rules_hbm.at[pl.ds(row0 + g * L, L), :]

    def blk_dst(b):
        return idx_v.at[pl.ds(b * L, L), :]

    for b in range(NBUF - 1):
        pltpu.async_copy(blk_src(b), blk_dst(b), sems[b])

    pltpu.sync_copy(table_hbm, tab_v.at[pl.ds(0, NUM_REL_RULES + 1)])

    def fuse_chunk(off, n):
        pltpu.sync_copy(g2l_hbm.at[pl.ds(off, n)], g2l_v.at[pl.ds(0, n)])
        h = n // 2

        for half in range(2):
            def body(i, carry):
                idx = g2l_v[pl.ds(half * h + i * L, L)]
                fchunk_v[pl.ds(i * L, L)] = plsc.load_gather(tab_v, [idx])
                return carry

            lax.fori_loop(0, h // L, body, 0)
            pltpu.sync_copy(fchunk_v.at[pl.ds(0, h)],
                            fused_sh.at[pl.ds(off + half * h, h)])

    @pl.when(s < NS - 1)
    def _():
        fuse_chunk(s * S_CHUNK, S_CHUNK)

    @pl.when(s == NS - 1)
    def _():
        fuse_chunk(LAST_OFF, LAST_N)

    plsc.subcore_barrier()
    pltpu.sync_copy(fused_sh, tab_v)

    UNROLL = 8

    def step(g, buf, queue_ahead):
        pltpu.make_async_copy(blk_src(g), blk_dst(buf), sems[buf]).wait()
        nbuf = (buf + NBUF - 1) % NBUF

        if queue_ahead:
            @pl.when(g + NBUF - 1 < GROUPS)
            def _():
                pltpu.async_copy(blk_src(g + NBUF - 1), blk_dst(nbuf),
                                 sems[nbuf])

        rows = iota + buf * L

        def hist(i, acc):
            l0 = i * UNROLL
            for u in range(UNROLL):
                idx = plsc.load_gather(idx_v, [rows, zeros + (l0 + u)])
                acc = acc + plsc.load_gather(tab_v, [idx])
            return acc

        acc = lax.fori_loop(0, HIST // UNROLL, hist,
                            jnp.zeros((L,), jnp.float32))
        out_v[pl.ds(g * L, L)] = acc

    def ring(i, carry):
        for b in range(NBUF):
            step(i * NBUF + b, b, True)
        return carry

    FULL = (GROUPS // NBUF) * NBUF
    lax.fori_loop(0, GROUPS // NBUF, ring, 0)
    for g in range(FULL, GROUPS):
        step(g, g % NBUF, False)

    pltpu.sync_copy(out_v, out_hbm.at[pl.ds(row0, ROWS_W)])


def kernel(rules, global_to_local, rules_table, bias):
    t_flat = jnp.minimum(rules_table[:, 0], jnp.float32(3.0e38))
    out = _run(rules, global_to_local, t_flat)
    return out.reshape(BATCH, 1) + bias


# speedup vs baseline: 1.1257x; 1.1257x over previous
import functools

import jax
import jax.numpy as jnp
from jax import lax
from jax.experimental import pallas as pl
from jax.experimental.pallas import tpu as pltpu
from jax.experimental.pallas import tpu_sc as plsc

LEN_RULES = 100000
NUM_REL_RULES = 50000
BATCH = 16384
HIST = 200

NC, NS, L = 2, 16, 16
NW = NC * NS

G_PAD = 100352
S_CHUNK = G_PAD // NS
S_HALF = S_CHUNK // 2
LAST_OFF = (NS - 1) * S_CHUNK
LAST_N = LEN_RULES - LAST_OFF
ROWS_W = BATCH // NW
GROUPS = ROWS_W // L
NBUF = 3

_mesh = plsc.VectorSubcoreMesh(core_axis_name="c", subcore_axis_name="s")
_params = pltpu.CompilerParams(needs_layout_passes=False)


@functools.partial(
    pl.kernel,
    out_type=jax.ShapeDtypeStruct((BATCH,), jnp.float32),
    mesh=_mesh,
    scratch_types=[
        pltpu.VMEM((G_PAD,), jnp.float32),
        pltpu.VMEM((NBUF * L, HIST), jnp.int32),
        pltpu.VMEM((S_CHUNK,), jnp.int32),
        pltpu.VMEM((S_HALF,), jnp.float32),
        pltpu.VMEM((ROWS_W,), jnp.float32),
        pltpu.VMEM_SHARED((G_PAD,), jnp.float32),
        [pltpu.SemaphoreType.DMA] * NBUF,
    ],
    compiler_params=_params,
)
def _run(rules_hbm, g2l_hbm, table_hbm, out_hbm,
         tab_v, idx_v, g2l_v, fchunk_v, out_v, fused_sh, sems):
    s = lax.axis_index("s")
    w = s * NC + lax.axis_index("c")
    row0 = w * ROWS_W
    iota = lax.iota(jnp.int32, L)
    zeros = jnp.zeros((L,), jnp.int32)

    def blk_src(g):
        return rules_hbm.at[pl.ds(row0 + g * L, L), :]

    def blk_dst(b):
        return idx_v.at[pl.ds(b * L, L), :]

    for b in range(NBUF - 1):
        pltpu.async_copy(blk_src(b), blk_dst(b), sems[b])

    pltpu.sync_copy(table_hbm, tab_v.at[pl.ds(0, NUM_REL_RULES + 1)])

    def fuse_chunk(off, n):
        pltpu.sync_copy(g2l_hbm.at[pl.ds(off, n)], g2l_v.at[pl.ds(0, n)])
        h = n // 2

        for half in range(2):
            def body(i, carry):
                idx = g2l_v[pl.ds(half * h + i * L, L)]
                fchunk_v[pl.ds(i * L, L)] = plsc.load_gather(tab_v, [idx])
                return carry

            lax.fori_loop(0, h // L, body, 0)
            pltpu.sync_copy(fchunk_v.at[pl.ds(0, h)],
                            fused_sh.at[pl.ds(off + half * h, h)])

    @pl.when(s < NS - 1)
    def _():
        fuse_chunk(s * S_CHUNK, S_CHUNK)

    @pl.when(s == NS - 1)
    def _():
        fuse_chunk(LAST_OFF, LAST_N)

    plsc.subcore_barrier()
    pltpu.sync_copy(fused_sh, tab_v)

    NFULL = HIST // L
    TAIL_OFF = HIST - L
    NDUP = NFULL * L - TAIL_OFF

    def step(g, buf, queue_ahead):
        pltpu.make_async_copy(blk_src(g), blk_dst(buf), sems[buf]).wait()
        nbuf = (buf + NBUF - 1) % NBUF

        if queue_ahead:
            @pl.when(g + NBUF - 1 < GROUPS)
            def _():
                pltpu.async_copy(blk_src(g + NBUF - 1), blk_dst(nbuf),
                                 sems[nbuf])

        for j in range(L):
            row = buf * L + j
            acc = jnp.zeros((L,), jnp.float32)
            for c in range(NFULL):
                idx16 = idx_v[row, pl.ds(c * L, L)]
                acc = acc + plsc.load_gather(tab_v, [idx16])
            idx16 = idx_v[row, pl.ds(TAIL_OFF, L)]
            vals = plsc.load_gather(tab_v, [idx16])
            acc = acc + jnp.where(iota < NDUP, 0.0, vals)
            s = jnp.sum(acc)
            plsc.store_scatter(out_v, [zeros + (g * L + j)],
                               jnp.zeros((L,), jnp.float32) + s,
                               mask=iota == 0)

    def ring(i, carry):
        for b in range(NBUF):
            step(i * NBUF + b, b, True)
        return carry

    FULL = (GROUPS // NBUF) * NBUF
    lax.fori_loop(0, GROUPS // NBUF, ring, 0)
    for g in range(FULL, GROUPS):
        step(g, g % NBUF, False)

    pltpu.sync_copy(out_v, out_hbm.at[pl.ds(row0, ROWS_W)])


def kernel(rules, global_to_local, rules_table, bias):
    t_flat = jnp.minimum(rules_table[:, 0], jnp.float32(3.0e38))
    out = _run(rules, global_to_local, t_flat)
    return out.reshape(BATCH, 1) + bias
